# baseline (speedup 1.0000x reference)
import functools

import numpy as np

import jax
import jax.numpy as jnp
from jax import lax
from jax.experimental import pallas as pl
from jax.experimental.pallas import tpu as pltpu
from jax.experimental.pallas import tpu_sc as plsc

NC = 2
NS = 16
C = 80


def _sc_agg_body(nloop, rows_per_sub, e_per_w,
                 xb_hbm, ei_hbm,
                 psum_hbm, pdeg_hbm,
                 src_v0, src_v1, src_v2,
                 dst_v0, dst_v1, dst_v2,
                 bf_v0, bf_v1, ff_v0, ff_v1, ones_v,
                 acc_sh, deg_sh,
                 sem_i0, sem_i1, sem_i2,
                 sem_g0, sem_g1,
                 sem_s0, sem_s1):
    cid = lax.axis_index("c")
    sid = lax.axis_index("s")
    wbase = (cid * NS + sid) * e_per_w
    src_v = (src_v0, src_v1, src_v2)
    dst_v = (dst_v0, dst_v1, dst_v2)
    bf_v = (bf_v0, bf_v1)
    ff_v = (ff_v0, ff_v1)
    sem_i = (sem_i0, sem_i1, sem_i2)
    sem_g = (sem_g0, sem_g1)
    sem_s = (sem_s0, sem_s1)
    d = ff_v0.shape[1]

    z16 = jnp.zeros((16,), jnp.float32)

    @pl.loop(0, C)
    def _(i):
        for k in range(d // 16):
            ff_v0[i, pl.ds(k * 16, 16)] = z16
        ones_v[i, pl.ds(0, 16)] = z16

    lo = sid * rows_per_sub
    nz = rows_per_sub // C
    for k in range(nz):
        pltpu.sync_copy(ff_v0, acc_sh.at[pl.ds(lo + k * C, C)])
        pltpu.sync_copy(ones_v, deg_sh.at[pl.ds(lo + k * C, C)])
    rem = rows_per_sub - nz * C
    if rem:
        pltpu.sync_copy(ff_v0.at[pl.ds(0, rem)],
                        acc_sh.at[pl.ds(lo + nz * C, rem)])
        pltpu.sync_copy(ones_v.at[pl.ds(0, rem)],
                        deg_sh.at[pl.ds(lo + nz * C, rem)])

    o16 = jnp.ones((16,), jnp.float32)

    @pl.loop(0, C)
    def _(i):
        ones_v[i, pl.ds(0, 16)] = o16

    plsc.subcore_barrier()

    def issue_idx(j, m):
        base = wbase + j * C
        pltpu.async_copy(ei_hbm.at[0, pl.ds(base, C)], src_v[m], sem_i[m])
        pltpu.async_copy(ei_hbm.at[1, pl.ds(base, C)], dst_v[m], sem_i[m])

    def wait_idx(j, m):
        base = wbase + j * C
        pltpu.make_async_copy(ei_hbm.at[0, pl.ds(base, C)], src_v[m],
                              sem_i[m]).wait()
        pltpu.make_async_copy(ei_hbm.at[1, pl.ds(base, C)], dst_v[m],
                              sem_i[m]).wait()

    def issue_gather(s, m):
        pltpu.async_copy(xb_hbm.at[src_v[m]], bf_v[s], sem_g[s])

    def wait_gather(s, m):
        pltpu.make_async_copy(xb_hbm.at[src_v[m]], bf_v[s],
                              sem_g[s]).wait()

    def convert(s):
        bf = bf_v[s]
        ff = ff_v[s]

        @pl.loop(0, C)
        def _(i):
            for g in range(d // 32):
                v = bf[i, pl.ds(32 * g, 32)]
                a, b = plsc.unpack(v, format=plsc.PackFormat.INTERLEAVED)
                ff[i, pl.ds(32 * g, 16)] = a
                ff[i, pl.ds(32 * g + 16, 16)] = b

    def issue_scat(s, m):
        pltpu.async_copy(ff_v[s], acc_sh.at[dst_v[m]], sem_s[s], add=True)
        pltpu.async_copy(ones_v, deg_sh.at[dst_v[m]], sem_s[s], add=True)

    def wait_scat(s, m):
        pltpu.make_async_copy(ff_v[s], acc_sh.at[dst_v[m]],
                              sem_s[s]).wait()
        pltpu.make_async_copy(ones_v, deg_sh.at[dst_v[m]],
                              sem_s[s]).wait()

    def stage(j, m, s, with_idx=True):
        wait_scat(s, (m + 1) % 3)
        if with_idx:
            issue_idx(j + 1, (m + 1) % 3)
        wait_idx(j, m)
        issue_gather(s, m)
        wait_gather(1 - s, (m + 2) % 3)
        convert(1 - s)
        issue_scat(1 - s, (m + 2) % 3)

    issue_idx(0, 0)
    wait_idx(0, 0)
    issue_gather(0, 0)
    issue_idx(1, 1)
    wait_idx(1, 1)
    issue_gather(1, 1)
    issue_idx(2, 2)
    wait_gather(0, 0)
    convert(0)
    issue_scat(0, 0)

    nsteady = (nloop - 5) // 6

    @pl.loop(0, nsteady)
    def _(t):
        j = 2 + t * 6
        for u in range(6):
            stage(j + u, (2 + u) % 3, u % 2)

    for j in range(nloop - 3, nloop):
        stage(j, j % 3, j % 2, with_idx=(j + 1 <= nloop - 1))

    s_l = (nloop - 1) % 2
    m_l = (nloop - 1) % 3
    wait_gather(s_l, m_l)
    convert(s_l)
    issue_scat(s_l, m_l)
    wait_scat(1 - s_l, (m_l + 2) % 3)
    wait_scat(s_l, m_l)

    plsc.subcore_barrier()

    pltpu.sync_copy(acc_sh.at[pl.ds(lo, rows_per_sub)],
                    psum_hbm.at[cid, pl.ds(lo, rows_per_sub)])
    pltpu.sync_copy(deg_sh.at[pl.ds(lo, rows_per_sub)],
                    pdeg_hbm.at[cid, pl.ds(lo, rows_per_sub)])


def _sc_aggregate(xb, edge_index):
    n, d = xb.shape
    e = edge_index.shape[1]
    assert e % (NC * C) == 0
    n_pad = -(-n // (NS * 8)) * (NS * 8)
    e_per_w = e // (NC * NS)
    assert e_per_w % C == 0
    nloop = e_per_w // C
    assert nloop >= 5 and (nloop - 5) % 6 == 0
    rows_per_sub = n_pad // NS
    assert rows_per_sub % 8 == 0 and (rows_per_sub % C) % 8 == 0

    mesh = plsc.VectorSubcoreMesh(core_axis_name="c", subcore_axis_name="s")
    body = functools.partial(_sc_agg_body, nloop, rows_per_sub, e_per_w)
    return pl.kernel(
        body,
        out_type=(jax.ShapeDtypeStruct((NC, n_pad, d), jnp.float32),
                  jax.ShapeDtypeStruct((NC, n_pad, 16), jnp.float32)),
        mesh=mesh,
        compiler_params=pltpu.CompilerParams(use_tc_tiling_on_sc=False,
                                             needs_layout_passes=False),
        scratch_types=(
            [pltpu.VMEM((C,), jnp.int32)] * 6
            + [pltpu.VMEM((C, d), jnp.bfloat16)] * 2
            + [pltpu.VMEM((C, d), jnp.float32)] * 2
            + [pltpu.VMEM((C, 16), jnp.float32)]
            + [pltpu.VMEM_SHARED((n_pad, d), jnp.float32),
               pltpu.VMEM_SHARED((n_pad, 16), jnp.float32)]
            + [pltpu.SemaphoreType.DMA] * 7
        ),
    )(xb, edge_index)


def _tc_body(p_ref, dp_ref, x_ref, wl_ref, wr_ref, bl_ref, lnw_ref, lnb_ref,
             o_ref):
    summed = p_ref[0] + p_ref[1]
    deg = dp_ref[0][:, :1] + dp_ref[1][:, :1]
    mean = summed / jnp.maximum(deg, 1.0)
    h = lax.dot_general(mean, wl_ref[...], (((1,), (1,)), ((), ())),
                        preferred_element_type=jnp.float32)
    h = h + lax.dot_general(x_ref[...], wr_ref[...], (((1,), (1,)), ((), ())),
                            preferred_element_type=jnp.float32)
    h = h + bl_ref[...]
    mu = jnp.mean(h, axis=-1, keepdims=True)
    hc = h - mu
    var = jnp.mean(hc * hc, axis=-1, keepdims=True)
    hn = hc * lax.rsqrt(var + 1e-5)
    o_ref[...] = jnp.maximum(hn * lnw_ref[...] + lnb_ref[...], 0.0)


def _tc_finish(psum, pdeg, x, W_l, b_l, W_r, ln_w, ln_b):
    n, d = x.shape
    blk = 2000
    grid = n // blk
    return pl.pallas_call(
        _tc_body,
        grid=(grid,),
        in_specs=[
            pl.BlockSpec((NC, blk, d), lambda i: (0, i, 0)),
            pl.BlockSpec((NC, blk, 16), lambda i: (0, i, 0)),
            pl.BlockSpec((blk, d), lambda i: (i, 0)),
            pl.BlockSpec((d, d), lambda i: (0, 0)),
            pl.BlockSpec((d, d), lambda i: (0, 0)),
            pl.BlockSpec((1, d), lambda i: (0, 0)),
            pl.BlockSpec((1, d), lambda i: (0, 0)),
            pl.BlockSpec((1, d), lambda i: (0, 0)),
        ],
        out_specs=pl.BlockSpec((blk, d), lambda i: (i, 0)),
        out_shape=jax.ShapeDtypeStruct((n, d), jnp.float32),
    )(psum, pdeg, x, W_l, W_r, b_l.reshape(1, d), ln_w.reshape(1, d),
      ln_b.reshape(1, d))


_PERM = np.arange(128).reshape(4, 2, 16).transpose(0, 2, 1).reshape(128)


def kernel(x, edge_index, W_l, b_l, W_r, ln_w, ln_b):
    xb = x[:, _PERM].astype(jnp.bfloat16)
    psum, pdeg = _sc_aggregate(xb, edge_index)
    return _tc_finish(psum, pdeg, x, W_l, b_l, W_r, ln_w, ln_b)


# speedup vs baseline: 1.9926x; 1.9926x over previous
import functools

import jax
import jax.numpy as jnp
from jax import lax
from jax.experimental import pallas as pl
from jax.experimental.pallas import tpu as pltpu
from jax.experimental.pallas import tpu_sc as plsc

NC = 2
NS = 16
C = 80


def _sc_agg_body(nloop, rows_per_sub, e_per_w,
                 x_hbm, ei_hbm,
                 psum_hbm, pdeg_hbm,
                 src_v0, src_v1, src_v2,
                 dst_v0, dst_v1, dst_v2,
                 rows_v0, rows_v1, rows_v2, ones_v,
                 acc_sh, deg_sh,
                 sem_i0, sem_i1, sem_i2,
                 sem_g0, sem_g1, sem_g2,
                 sem_s0, sem_s1, sem_s2):
    cid = lax.axis_index("c")
    sid = lax.axis_index("s")
    wbase = (cid * NS + sid) * e_per_w
    src_v = (src_v0, src_v1, src_v2)
    dst_v = (dst_v0, dst_v1, dst_v2)
    rows_v = (rows_v0, rows_v1, rows_v2)
    sem_i = (sem_i0, sem_i1, sem_i2)
    sem_g = (sem_g0, sem_g1, sem_g2)
    sem_s = (sem_s0, sem_s1, sem_s2)

    z16 = jnp.zeros((16,), jnp.float32)

    @pl.loop(0, C)
    def _(i):
        for k in range(rows_v0.shape[1] // 16):
            rows_v0[i, pl.ds(k * 16, 16)] = z16
        ones_v[i, pl.ds(0, 16)] = z16

    lo = sid * rows_per_sub
    nz = rows_per_sub // C
    for k in range(nz):
        pltpu.sync_copy(rows_v0, acc_sh.at[pl.ds(lo + k * C, C)])
        pltpu.sync_copy(ones_v, deg_sh.at[pl.ds(lo + k * C, C)])
    rem = rows_per_sub - nz * C
    if rem:
        pltpu.sync_copy(rows_v0.at[pl.ds(0, rem)],
                        acc_sh.at[pl.ds(lo + nz * C, rem)])
        pltpu.sync_copy(ones_v.at[pl.ds(0, rem)],
                        deg_sh.at[pl.ds(lo + nz * C, rem)])

    o16 = jnp.ones((16,), jnp.float32)

    @pl.loop(0, C)
    def _(i):
        ones_v[i, pl.ds(0, 16)] = o16

    plsc.subcore_barrier()

    def issue_idx(j, s):
        base = wbase + j * C
        pltpu.async_copy(ei_hbm.at[0, pl.ds(base, C)], src_v[s], sem_i[s])
        pltpu.async_copy(ei_hbm.at[1, pl.ds(base, C)], dst_v[s], sem_i[s])

    def wait_idx(j, s):
        base = wbase + j * C
        pltpu.make_async_copy(ei_hbm.at[0, pl.ds(base, C)], src_v[s],
                              sem_i[s]).wait()
        pltpu.make_async_copy(ei_hbm.at[1, pl.ds(base, C)], dst_v[s],
                              sem_i[s]).wait()

    def issue_gather(s):
        pltpu.async_copy(x_hbm.at[src_v[s]], rows_v[s], sem_g[s])

    def wait_gather(s):
        pltpu.make_async_copy(x_hbm.at[src_v[s]], rows_v[s],
                              sem_g[s]).wait()

    def issue_scat(s):
        pltpu.async_copy(rows_v[s], acc_sh.at[dst_v[s]], sem_s[s], add=True)
        pltpu.async_copy(ones_v, deg_sh.at[dst_v[s]], sem_s[s], add=True)

    def wait_scat(s):
        pltpu.make_async_copy(rows_v[s], acc_sh.at[dst_v[s]],
                              sem_s[s]).wait()
        pltpu.make_async_copy(ones_v, deg_sh.at[dst_v[s]],
                              sem_s[s]).wait()

    def stage(j, s, with_idx=True):
        wait_scat((s + 1) % 3)
        if with_idx:
            issue_idx(j + 1, (s + 1) % 3)
        wait_idx(j, s)
        issue_gather(s)
        wait_gather((s + 2) % 3)
        issue_scat((s + 2) % 3)

    issue_idx(0, 0)
    wait_idx(0, 0)
    issue_gather(0)
    issue_idx(1, 1)
    wait_idx(1, 1)
    issue_gather(1)
    issue_idx(2, 2)
    wait_gather(0)
    issue_scat(0)

    nsteady = (nloop - 5) // 3

    @pl.loop(0, nsteady)
    def _(t):
        j = 2 + t * 3
        stage(j, 2)
        stage(j + 1, 0)
        stage(j + 2, 1)

    stage(nloop - 3, 2)
    stage(nloop - 2, 0)
    stage(nloop - 1, 1, with_idx=False)
    wait_gather(1)
    issue_scat(1)
    wait_scat(0)
    wait_scat(1)

    plsc.subcore_barrier()

    pltpu.sync_copy(acc_sh.at[pl.ds(lo, rows_per_sub)],
                    psum_hbm.at[cid, pl.ds(lo, rows_per_sub)])
    pltpu.sync_copy(deg_sh.at[pl.ds(lo, rows_per_sub)],
                    pdeg_hbm.at[cid, pl.ds(lo, rows_per_sub)])


def _sc_aggregate(x, edge_index):
    n, d = x.shape
    e = edge_index.shape[1]
    assert e % (NC * C) == 0
    n_pad = -(-n // (NS * 8)) * (NS * 8)
    e_per_w = e // (NC * NS)
    assert e_per_w % C == 0
    nloop = e_per_w // C
    assert nloop >= 5 and (nloop - 5) % 3 == 0
    rows_per_sub = n_pad // NS
    assert rows_per_sub % 8 == 0 and (rows_per_sub % C) % 8 == 0

    mesh = plsc.VectorSubcoreMesh(core_axis_name="c", subcore_axis_name="s")
    body = functools.partial(_sc_agg_body, nloop, rows_per_sub, e_per_w)
    return pl.kernel(
        body,
        out_type=(jax.ShapeDtypeStruct((NC, n_pad, d), jnp.float32),
                  jax.ShapeDtypeStruct((NC, n_pad, 16), jnp.float32)),
        mesh=mesh,
        compiler_params=pltpu.CompilerParams(use_tc_tiling_on_sc=False),
        scratch_types=(
            [pltpu.VMEM((C,), jnp.int32)] * 6
            + [pltpu.VMEM((C, d), jnp.float32)] * 3
            + [pltpu.VMEM((C, 16), jnp.float32)]
            + [pltpu.VMEM_SHARED((n_pad, d), jnp.float32),
               pltpu.VMEM_SHARED((n_pad, 16), jnp.float32)]
            + [pltpu.SemaphoreType.DMA] * 9
        ),
    )(x, edge_index)


def _tc_body(p_ref, dp_ref, x_ref, wl_ref, wr_ref, bl_ref, lnw_ref, lnb_ref,
             o_ref):
    summed = p_ref[0] + p_ref[1]
    deg = dp_ref[0][:, :1] + dp_ref[1][:, :1]
    mean = summed / jnp.maximum(deg, 1.0)
    h = lax.dot_general(mean, wl_ref[...], (((1,), (1,)), ((), ())),
                        preferred_element_type=jnp.float32)
    h = h + lax.dot_general(x_ref[...], wr_ref[...], (((1,), (1,)), ((), ())),
                            preferred_element_type=jnp.float32)
    h = h + bl_ref[...]
    mu = jnp.mean(h, axis=-1, keepdims=True)
    hc = h - mu
    var = jnp.mean(hc * hc, axis=-1, keepdims=True)
    hn = hc * lax.rsqrt(var + 1e-5)
    o_ref[...] = jnp.maximum(hn * lnw_ref[...] + lnb_ref[...], 0.0)


def _tc_finish(psum, pdeg, x, W_l, b_l, W_r, ln_w, ln_b):
    n, d = x.shape
    blk = 2000
    grid = n // blk
    return pl.pallas_call(
        _tc_body,
        grid=(grid,),
        in_specs=[
            pl.BlockSpec((NC, blk, d), lambda i: (0, i, 0)),
            pl.BlockSpec((NC, blk, 16), lambda i: (0, i, 0)),
            pl.BlockSpec((blk, d), lambda i: (i, 0)),
            pl.BlockSpec((d, d), lambda i: (0, 0)),
            pl.BlockSpec((d, d), lambda i: (0, 0)),
            pl.BlockSpec((1, d), lambda i: (0, 0)),
            pl.BlockSpec((1, d), lambda i: (0, 0)),
            pl.BlockSpec((1, d), lambda i: (0, 0)),
        ],
        out_specs=pl.BlockSpec((blk, d), lambda i: (i, 0)),
        out_shape=jax.ShapeDtypeStruct((n, d), jnp.float32),
    )(psum, pdeg, x, W_l, W_r, b_l.reshape(1, d), ln_w.reshape(1, d),
      ln_b.reshape(1, d))


def kernel(x, edge_index, W_l, b_l, W_r, ln_w, ln_b):
    psum, pdeg = _sc_aggregate(x, edge_index)
    return _tc_finish(psum, pdeg, x, W_l, b_l, W_r, ln_w, ln_b)
